# baseline (speedup 1.0000x reference)
import functools

import jax
import jax.numpy as jnp
from jax.experimental import pallas as pl
from jax.experimental.pallas import tpu as pltpu

_B, _NS, _NT, _D = 4, 2048, 2048, 64
_RP2 = 0.25
_RN = 1.0
_NEG = -1e30


def _fused_kernel(sf_ref, tf_ref, sxyz_ref, txyz_ref, w_ref, out_ref):
    b = pl.program_id(0)

    sf = sf_ref[0]
    tf = tf_ref[0]
    sxyz = sxyz_ref[0]
    txyz = txyz_ref[0]
    w = w_ref[...]

    r = jax.lax.broadcasted_iota(jnp.int32, (_D, _D), 0)
    c = jax.lax.broadcasted_iota(jnp.int32, (_D, _D), 1)
    wt = jnp.where(r <= c, w, 0.0)
    ws = wt + wt.T

    sfw = jnp.dot(sf, ws, preferred_element_type=jnp.float32)
    logits = jax.lax.dot_general(
        sfw.astype(jnp.bfloat16), tf.astype(jnp.bfloat16),
        (((1,), (1,)), ((), ())),
        preferred_element_type=jnp.float32)

    a2h = 0.5 * jnp.sum(sxyz * sxyz, axis=1, keepdims=True)
    b2h = 0.5 * jnp.sum(txyz * txyz, axis=1, keepdims=True)
    aug_l = jnp.concatenate(
        [sxyz, a2h, jnp.ones((sxyz.shape[0], 1), jnp.float32)], axis=1)
    aug_r = jnp.concatenate(
        [txyz, -jnp.ones((txyz.shape[0], 1), jnp.float32), -b2h], axis=1)
    m2 = jax.lax.dot_general(
        aug_l, aug_r, (((1,), (1,)), ((), ())),
        preferred_element_type=jnp.float32)

    m2max = jnp.max(m2, axis=1, keepdims=True)
    pos = jnp.max(jnp.where(m2 == m2max, logits, _NEG),
                  axis=1, keepdims=True)
    e = jnp.exp(jnp.where(m2 > -0.5 * _RN * _RN, _NEG, logits))
    srow = jnp.sum(e, axis=1, keepdims=True)
    srow = srow + jnp.where(m2max > -0.5 * _RN * _RN, jnp.exp(pos), 0.0)
    lse = jnp.log(srow)
    loss_per = lse - pos

    valid = m2max > -0.5 * _RP2
    psum = jnp.sum(jnp.where(valid, loss_per, 0.0))
    pcnt = jnp.sum(valid.astype(jnp.float32))

    @pl.when(b == 0)
    def _():
        out_ref[...] = jnp.zeros_like(out_ref)

    out_ref[...] += psum / (pcnt * _B)


@functools.partial(jax.jit, static_argnames=("interpret",))
def kernel(src_feat, tgt_feat, src_xyz, tgt_xyz, W, interpret=False):
    out = pl.pallas_call(
        _fused_kernel,
        grid=(_B,),
        in_specs=[
            pl.BlockSpec((1, _NS, _D), lambda b: (b, 0, 0)),
            pl.BlockSpec((1, _NT, _D), lambda b: (b, 0, 0)),
            pl.BlockSpec((1, _NS, 3), lambda b: (b, 0, 0)),
            pl.BlockSpec((1, _NT, 3), lambda b: (b, 0, 0)),
            pl.BlockSpec((_D, _D), lambda b: (0, 0)),
        ],
        out_specs=pl.BlockSpec((1, 1, 128), lambda b: (0, 0, 0)),
        out_shape=jax.ShapeDtypeStruct((1, 1, 128), jnp.float32),
        compiler_params=pltpu.CompilerParams(
            dimension_semantics=("arbitrary",)),
        interpret=interpret,
    )(src_feat, tgt_feat, src_xyz, tgt_xyz, W)

    return out[0, 0, 0]


# speedup vs baseline: 1.0003x; 1.0003x over previous
import functools

import jax
import jax.numpy as jnp
from jax.experimental import pallas as pl
from jax.experimental.pallas import tpu as pltpu

_B, _NS, _NT, _D = 4, 2048, 2048, 64
_RP2 = 0.25
_RN = 1.0
_NEG = -1e30


def _fused_kernel(sf_ref, tf_ref, sxyz_ref, txyz_ref, w_ref, out_ref):
    b = pl.program_id(0)

    sf = sf_ref[0]
    tf = tf_ref[0]
    sxyz = sxyz_ref[0]
    txyz = txyz_ref[0]
    w = w_ref[...]

    r = jax.lax.broadcasted_iota(jnp.int32, (_D, _D), 0)
    c = jax.lax.broadcasted_iota(jnp.int32, (_D, _D), 1)
    wt = jnp.where(r <= c, w, 0.0)
    ws = wt + wt.T

    sfw = jnp.dot(sf, ws, preferred_element_type=jnp.float32)
    logits = jax.lax.dot_general(
        sfw, tf, (((1,), (1,)), ((), ())),
        preferred_element_type=jnp.float32)

    a2h = 0.5 * jnp.sum(sxyz * sxyz, axis=1, keepdims=True)
    b2h = 0.5 * jnp.sum(txyz * txyz, axis=1, keepdims=True)
    zl = jnp.zeros((sxyz.shape[0], 3), jnp.float32)
    aug_l = jnp.concatenate(
        [sxyz, a2h, jnp.ones((sxyz.shape[0], 1), jnp.float32), zl], axis=1)
    aug_r = jnp.concatenate(
        [txyz, -jnp.ones((txyz.shape[0], 1), jnp.float32), -b2h,
         jnp.zeros((txyz.shape[0], 3), jnp.float32)], axis=1)
    m2 = jax.lax.dot_general(
        aug_l, aug_r, (((1,), (1,)), ((), ())),
        preferred_element_type=jnp.float32)

    m2max = jnp.max(m2, axis=1, keepdims=True)
    pos = jnp.max(jnp.where(m2 == m2max, logits, jnp.float32(_NEG)),
                  axis=1, keepdims=True)
    e = jnp.exp(jnp.where(m2 > -0.5 * _RN * _RN, jnp.float32(_NEG), logits))
    srow = jnp.sum(e, axis=1, keepdims=True)
    srow = srow + jnp.where(m2max > -0.5 * _RN * _RN, jnp.exp(pos), 0.0)
    lse = jnp.log(srow)
    loss_per = lse - pos

    valid = m2max > -0.5 * _RP2
    psum = jnp.sum(jnp.where(valid, loss_per, 0.0))
    pcnt = jnp.sum(valid.astype(jnp.float32))

    @pl.when(b == 0)
    def _():
        out_ref[...] = jnp.zeros_like(out_ref)

    out_ref[...] += psum / (pcnt * _B)


@functools.partial(jax.jit, static_argnames=("interpret",))
def kernel(src_feat, tgt_feat, src_xyz, tgt_xyz, W, interpret=False):
    out = pl.pallas_call(
        _fused_kernel,
        grid=(_B,),
        in_specs=[
            pl.BlockSpec((1, _NS, _D), lambda b: (b, 0, 0)),
            pl.BlockSpec((1, _NT, _D), lambda b: (b, 0, 0)),
            pl.BlockSpec((1, _NS, 3), lambda b: (b, 0, 0)),
            pl.BlockSpec((1, _NT, 3), lambda b: (b, 0, 0)),
            pl.BlockSpec((_D, _D), lambda b: (0, 0)),
        ],
        out_specs=pl.BlockSpec((1, 1, 128), lambda b: (0, 0, 0)),
        out_shape=jax.ShapeDtypeStruct((1, 1, 128), jnp.float32),
        compiler_params=pltpu.CompilerParams(
            dimension_semantics=("arbitrary",)),
        interpret=interpret,
    )(src_feat, tgt_feat, src_xyz, tgt_xyz, W)

    return out[0, 0, 0]


# speedup vs baseline: 1.0227x; 1.0224x over previous
import functools

import jax
import jax.numpy as jnp
from jax.experimental import pallas as pl
from jax.experimental.pallas import tpu as pltpu

_B, _NS, _NT, _D = 4, 2048, 2048, 64
_RP2 = 0.25
_RN = 1.0
_NEG = -1e30


_CHUNK = 512


def _fused_kernel(sf_ref, tf_ref, sxyz_ref, txyz_ref, w_ref, out_ref):
    b = pl.program_id(0)

    tf = tf_ref[0]
    txyz = txyz_ref[0]
    w = w_ref[...]

    r = jax.lax.broadcasted_iota(jnp.int32, (_D, _D), 0)
    c = jax.lax.broadcasted_iota(jnp.int32, (_D, _D), 1)
    wt = jnp.where(r <= c, w, 0.0)
    ws = wt + wt.T

    b2h = 0.5 * jnp.sum(txyz * txyz, axis=1, keepdims=True)
    aug_r = jnp.concatenate(
        [txyz, -jnp.ones((txyz.shape[0], 1), jnp.float32), -b2h], axis=1)

    psum = jnp.float32(0.0)
    pcnt = jnp.float32(0.0)
    for i in range(_NS // _CHUNK):
        sf = sf_ref[0, pl.ds(i * _CHUNK, _CHUNK), :]
        sxyz = sxyz_ref[0, pl.ds(i * _CHUNK, _CHUNK), :]

        sfw = jnp.dot(sf, ws, preferred_element_type=jnp.float32)
        logits = jax.lax.dot_general(
            sfw, tf, (((1,), (1,)), ((), ())),
            preferred_element_type=jnp.float32)

        a2h = 0.5 * jnp.sum(sxyz * sxyz, axis=1, keepdims=True)
        aug_l = jnp.concatenate(
            [sxyz, a2h, jnp.ones((sxyz.shape[0], 1), jnp.float32)], axis=1)
        m2 = jax.lax.dot_general(
            aug_l, aug_r, (((1,), (1,)), ((), ())),
            preferred_element_type=jnp.float32)

        m2max = jnp.max(m2, axis=1, keepdims=True)
        pos = jnp.max(jnp.where(m2 == m2max, logits, jnp.float32(_NEG)),
                      axis=1, keepdims=True)
        e = jnp.exp(jnp.where(m2 > -0.5 * _RN * _RN, jnp.float32(_NEG), logits))
        srow = jnp.sum(e, axis=1, keepdims=True)
        srow = srow + jnp.where(m2max > -0.5 * _RN * _RN, jnp.exp(pos), 0.0)
        lse = jnp.log(srow)
        loss_per = lse - pos

        valid = m2max > -0.5 * _RP2
        psum += jnp.sum(jnp.where(valid, loss_per, 0.0))
        pcnt += jnp.sum(valid.astype(jnp.float32))

    @pl.when(b == 0)
    def _():
        out_ref[...] = jnp.zeros_like(out_ref)

    out_ref[...] += psum / (pcnt * _B)


@functools.partial(jax.jit, static_argnames=("interpret",))
def kernel(src_feat, tgt_feat, src_xyz, tgt_xyz, W, interpret=False):
    out = pl.pallas_call(
        _fused_kernel,
        grid=(_B,),
        in_specs=[
            pl.BlockSpec((1, _NS, _D), lambda b: (b, 0, 0)),
            pl.BlockSpec((1, _NT, _D), lambda b: (b, 0, 0)),
            pl.BlockSpec((1, _NS, 3), lambda b: (b, 0, 0)),
            pl.BlockSpec((1, _NT, 3), lambda b: (b, 0, 0)),
            pl.BlockSpec((_D, _D), lambda b: (0, 0)),
        ],
        out_specs=pl.BlockSpec((1, 1, 128), lambda b: (0, 0, 0)),
        out_shape=jax.ShapeDtypeStruct((1, 1, 128), jnp.float32),
        compiler_params=pltpu.CompilerParams(
            dimension_semantics=("arbitrary",)),
        interpret=interpret,
    )(src_feat, tgt_feat, src_xyz, tgt_xyz, W)

    return out[0, 0, 0]
